# baseline (speedup 1.0000x reference)
import jax
import jax.numpy as jnp
from jax import lax
from jax.experimental import pallas as pl
from jax.experimental.pallas import tpu as pltpu
from jax.experimental.pallas import tpu_sc as plsc

N = 10000
NP = 10240
KN = 32
KP = 15
CIN = 128
COUT = 128
BQ = 256
SIGMA = 1.0
PREV_GRID = 0.1
RADIUS = 2.5 * SIGMA * PREV_GRID
POINT_INFLUENCE = PREV_GRID * SIGMA
BN_EPS = 1e-5

NW = 32
QPW = NP // NW


def _d2_body(qb_ref, sbt_ref, q2_ref, s2_ref, d2_ref):
    dot = lax.dot_general(qb_ref[...], sbt_ref[...], (((1,), (0,)), ((), ())),
                          preferred_element_type=jnp.float32)
    d2_ref[...] = q2_ref[...] + s2_ref[...] - 2.0 * dot


def _d2mat(qb, sbt, sq):
    return pl.pallas_call(
        _d2_body,
        grid=(NP // BQ,),
        in_specs=[
            pl.BlockSpec((BQ, 3), lambda i: (i, 0)),
            pl.BlockSpec((3, NP), lambda i: (0, 0)),
            pl.BlockSpec((BQ, 1), lambda i: (i, 0)),
            pl.BlockSpec((1, NP), lambda i: (0, 0)),
        ],
        out_specs=pl.BlockSpec((BQ, NP), lambda i: (i, 0)),
        out_shape=jax.ShapeDtypeStruct((NP, NP), jnp.float32),
    )(qb, sbt, sq[:, None], sq[None, :])


def _bf16_round(f):
    u = lax.bitcast_convert_type(f, jnp.int32)
    bias = 0x7FFF + (lax.shift_right_logical(u, 16) & 1)
    u = (u + bias) & jnp.int32(-65536)
    return lax.bitcast_convert_type(u, jnp.float32)


def _nr_sqrt(sq):
    i = lax.bitcast_convert_type(sq, jnp.int32)
    i = 0x5F3759DF - lax.shift_right_logical(i, 1)
    y = lax.bitcast_convert_type(i, jnp.float32)
    for _ in range(3):
        y = y * (1.5 - 0.5 * sq * y * y)
    return sq * y


ROWW = 144
NG = NP // 16
SEGA = 272
CAP2 = 176
LO = -0.07
NB_BINS = 64
DELTA = (RADIUS * RADIUS - LO) / NB_BINS
R2 = RADIUS * RADIUS


def _sc_body(d2_hbm, xfull_hbm, qpos_hbm, kp_hbm, nbr_hbm, wsum_hbm,
             qpos_v, kp_v, d2r_v, cand_v, candi_v, c2v_v, c2i_v, hist_v,
             idx_v, rows_v, wbuf_v, acc_v, nbub_v, semd, semg):
    c = lax.axis_index("c")
    s = lax.axis_index("s")
    wid = s * 2 + c
    base = wid * QPW
    pltpu.sync_copy(qpos_hbm.at[pl.ds(base, QPW)], qpos_v)
    pltpu.sync_copy(kp_hbm, kp_v)
    zero16 = jnp.zeros((16,), jnp.float32)
    for cc in range(8):
        acc_v[KP, pl.ds(cc * 16, 16)] = zero16
    inv_pi = jnp.float32(POINT_INFLUENCE)
    lane16 = jnp.arange(16, dtype=jnp.int32)
    r2f = jnp.float32(R2)
    inf16 = jnp.full((16,), jnp.inf, jnp.float32)

    pltpu.async_copy(d2_hbm.at[base], d2r_v.at[pl.ds(0, NP)], semd)

    def qbody(m, carry):
        buf = lax.rem(m, 2)

        @pl.when(m < QPW)
        def _select():
            doff = buf * NP
            pltpu.make_async_copy(d2_hbm.at[base],
                                  d2r_v.at[pl.ds(0, NP)], semd).wait()
            mn = jnp.minimum(m + 1, QPW - 1)
            pltpu.async_copy(d2_hbm.at[base + mn],
                             d2r_v.at[pl.ds((1 - buf) * NP, NP)], semd)

            def abody(g, offs):
                o = list(offs)
                for q in range(4):
                    gg = q * (NG // 4) + g
                    v = d2r_v[pl.ds(doff + gg * 16, 16)]
                    mask = v <= r2f
                    plsc.store_compressed(
                        cand_v.at[pl.ds(q * SEGA + o[q], 16)], v, mask=mask)
                    iv = lane16 + gg * 16
                    plsc.store_compressed(
                        candi_v.at[pl.ds(q * SEGA + o[q], 16)], iv, mask=mask)
                    cnt = plsc.all_reduce_population_count(mask)
                    cnt = cnt[0] if cnt.ndim else cnt
                    o[q] = jnp.minimum(o[q] + cnt, SEGA - 16)
                return tuple(o)

            offs = lax.fori_loop(0, NG // 4, abody,
                                 (jnp.int32(0),) * 4, unroll=2)
            for q in range(4):
                cand_v[pl.ds(q * SEGA + offs[q], 16)] = inf16

            for hh in range(NB_BINS // 16):
                hist_v[pl.ds(hh * 16, 16)] = jnp.zeros((16,), jnp.int32)

            for q in range(4):
                offq = offs[q]

                def bbody(g, carry2, _q=q, _offq=offq):
                    v = cand_v[pl.ds(_q * SEGA + g * 16, 16)]
                    valid = (lane16 + g * 16) < _offq
                    bi = ((v - jnp.float32(LO)) * jnp.float32(1.0 / DELTA))
                    bi = jnp.clip(bi.astype(jnp.int32), 0, NB_BINS - 1)
                    bi = jnp.where(valid, bi, NB_BINS - 1)
                    plsc.addupdate_scatter(hist_v, [bi],
                                           jnp.ones((16,), jnp.int32))
                    return carry2

                lax.fori_loop(0, (offq + 15) // 16, bbody, 0)

            cum = jnp.int32(0)
            bstar = jnp.int32(NB_BINS)
            for hh in range(NB_BINS // 16):
                ch = plsc.cumsum(hist_v[pl.ds(hh * 16, 16)]) + cum
                bh = jnp.where(ch >= KN, lane16 + hh * 16, NB_BINS)
                bstar = jnp.minimum(bstar, jnp.min(bh))
                cum = ch[15]

            off2 = jnp.int32(0)
            for q in range(4):
                offq = offs[q]

                def cbody(g, o2, _q=q, _offq=offq):
                    v = cand_v[pl.ds(_q * SEGA + g * 16, 16)]
                    iv2 = candi_v[pl.ds(_q * SEGA + g * 16, 16)]
                    valid = (lane16 + g * 16) < _offq
                    bi = ((v - jnp.float32(LO)) * jnp.float32(1.0 / DELTA))
                    bi = jnp.clip(bi.astype(jnp.int32), 0, NB_BINS - 1)
                    m2 = (bi <= bstar) & valid
                    plsc.store_compressed(c2v_v.at[pl.ds(o2, 16)], v,
                                          mask=m2)
                    plsc.store_compressed(c2i_v.at[pl.ds(o2, 16)], iv2,
                                          mask=m2)
                    cnt = plsc.all_reduce_population_count(m2)
                    cnt = cnt[0] if cnt.ndim else cnt
                    return jnp.minimum(o2 + cnt, CAP2 - 32)

                off2 = lax.fori_loop(0, (offq + 15) // 16, cbody, off2)

            c2v_v[pl.ds(off2, 16)] = inf16
            nv2 = (off2 + 15) // 16

            lastm = jnp.full((16,), -jnp.inf, jnp.float32)
            lasti = jnp.full((16,), -1, jnp.int32)
            nb0 = jnp.full((16,), N, jnp.int32)
            nb1 = jnp.full((16,), N, jnp.int32)
            for t in range(KN):
                def dbody(g, mm):
                    mval, midx = mm
                    v = c2v_v[pl.ds(g * 16, 16)]
                    ix = c2i_v[pl.ds(g * 16, 16)]
                    elig = (v > lastm) | ((v == lastm) & (ix > lasti))
                    bet = elig & ((v < mval) | ((v == mval) & (ix < midx)))
                    return (jnp.where(bet, v, mval),
                            jnp.where(bet, ix, midx))

                mval, midx = lax.fori_loop(
                    0, nv2, dbody,
                    (inf16, jnp.full((16,), NP, jnp.int32)))
                m_sc = jnp.min(mval)
                i_sc = jnp.min(jnp.where(mval == m_sc, midx, NP))
                nb_sc = jnp.where(m_sc <= r2f, i_sc, N)
                if t < 16:
                    nb0 = jnp.where(lane16 == t, nb_sc, nb0)
                else:
                    nb1 = jnp.where(lane16 == (t - 16), nb_sc, nb1)
                lastm = jnp.full((16,), m_sc, jnp.float32)
                lasti = jnp.full((16,), i_sc, jnp.int32)

            nbub_v[pl.ds(0, 16)] = nb0
            nbub_v[pl.ds(16, 16)] = nb1
            pltpu.sync_copy(nbub_v, nbr_hbm.at[base + m])
            idx_v[pl.ds(buf * KN, 16)] = nb0
            idx_v[pl.ds(buf * KN + 16, 16)] = nb1

        @pl.when(m > 0)
        def _agg():
            aoff = (1 - buf) * KN
            pltpu.make_async_copy(
                xfull_hbm.at[idx_v.at[pl.ds(aoff, KN)]],
                rows_v.at[pl.ds(aoff, KN)], semg).wait()
            qrow = qpos_v[m - 1, pl.ds(0, 16)]
            qx = qrow[0]
            qy = qrow[1]
            qz = qrow[2]
            kpx = kp_v[0, :]
            kpy = kp_v[1, :]
            kpz = kp_v[2, :]

            def wbody(n, carry2):
                pv = rows_v[aoff + n, pl.ds(CIN, 16)]
                ax = (pv[0] - qx) - kpx
                ay = (pv[1] - qy) - kpy
                az = (pv[2] - qz) - kpz
                sq2 = ax * ax + ay * ay + az * az
                sr = _nr_sqrt(sq2)
                w = jnp.maximum(1.0 - sr / inv_pi, 0.0)
                wbuf_v[n, pl.ds(0, 16)] = _bf16_round(w)
                return carry2

            lax.fori_loop(0, KN, wbody, 0, unroll=4)

            for kg in range(3):
                def nbody(n, accs):
                    rr = [rows_v[aoff + n, pl.ds(cc * 16, 16)]
                          for cc in range(8)]
                    wv = wbuf_v[n, pl.ds(0, 16)]
                    out = list(accs)
                    for j in range(5):
                        wk = wv[kg * 5 + j]
                        for cc in range(8):
                            out[j * 8 + cc] = out[j * 8 + cc] + wk * rr[cc]
                    return tuple(out)

                accs = lax.fori_loop(0, KN, nbody,
                                     tuple(jnp.zeros((16,), jnp.float32)
                                           for _ in range(40)),
                                     unroll=2)
                for j in range(5):
                    for cc in range(8):
                        acc_v[kg * 5 + j, pl.ds(cc * 16, 16)] = \
                            accs[j * 8 + cc]
            pltpu.sync_copy(acc_v, wsum_hbm.at[base + m - 1])

        @pl.when(m < QPW)
        def _issue_gather():
            pltpu.async_copy(
                xfull_hbm.at[idx_v.at[pl.ds(buf * KN, KN)]],
                rows_v.at[pl.ds(buf * KN, KN)], semg)

        return carry

    lax.fori_loop(0, QPW + 1, qbody, 0)
    pltpu.make_async_copy(d2_hbm.at[base],
                          d2r_v.at[pl.ds(0, NP)], semd).wait()


def _sc_aggregate(d2, xfull, qpos, kp):
    mesh = plsc.VectorSubcoreMesh(core_axis_name="c", subcore_axis_name="s")
    f = pl.kernel(
        _sc_body,
        out_type=[
            jax.ShapeDtypeStruct((NP, KN), jnp.int32),
            jax.ShapeDtypeStruct((NP, KP + 1, CIN), jnp.float32),
        ],
        mesh=mesh,
        compiler_params=pltpu.CompilerParams(use_tc_tiling_on_sc=False,
                                             needs_layout_passes=False),
        scratch_types=[
            pltpu.VMEM((QPW, 16), jnp.float32),
            pltpu.VMEM((3, 16), jnp.float32),
            pltpu.VMEM((2 * NP,), jnp.float32),
            pltpu.VMEM((4 * SEGA,), jnp.float32),
            pltpu.VMEM((4 * SEGA,), jnp.int32),
            pltpu.VMEM((CAP2,), jnp.float32),
            pltpu.VMEM((CAP2,), jnp.int32),
            pltpu.VMEM((NB_BINS,), jnp.int32),
            pltpu.VMEM((2 * KN,), jnp.int32),
            pltpu.VMEM((2 * KN, ROWW), jnp.float32),
            pltpu.VMEM((KN, 16), jnp.float32),
            pltpu.VMEM((KP + 1, CIN), jnp.float32),
            pltpu.VMEM((KN,), jnp.int32),
            pltpu.SemaphoreType.DMA,
            pltpu.SemaphoreType.DMA,
        ],
    )
    return f(d2, xfull, qpos, kp)


def _mm_body(ws_ref, wf_ref, out_ref, sum_ref, sq_ref):
    i = pl.program_id(0)
    a = ws_ref[...].astype(jnp.bfloat16)
    o = lax.dot_general(a, wf_ref[...], (((1,), (0,)), ((), ())),
                        preferred_element_type=jnp.float32)
    out_ref[...] = o
    rowid = i * BQ + lax.broadcasted_iota(jnp.int32, (BQ, 1), 0)
    om = jnp.where(rowid < N, o, 0.0)
    ps = jnp.sum(om, axis=0, keepdims=True)
    pq = jnp.sum(om * om, axis=0, keepdims=True)

    @pl.when(i == 0)
    def _():
        sum_ref[...] = jnp.zeros_like(sum_ref)
        sq_ref[...] = jnp.zeros_like(sq_ref)

    sum_ref[...] += jnp.broadcast_to(ps, (8, COUT))
    sq_ref[...] += jnp.broadcast_to(pq, (8, COUT))


def _mm_stats(wsum2d, wf):
    return pl.pallas_call(
        _mm_body,
        grid=(NP // BQ,),
        in_specs=[
            pl.BlockSpec((BQ, (KP + 1) * CIN), lambda i: (i, 0)),
            pl.BlockSpec(((KP + 1) * CIN, COUT), lambda i: (0, 0)),
        ],
        out_specs=[
            pl.BlockSpec((BQ, COUT), lambda i: (i, 0)),
            pl.BlockSpec((8, COUT), lambda i: (0, 0)),
            pl.BlockSpec((8, COUT), lambda i: (0, 0)),
        ],
        out_shape=[
            jax.ShapeDtypeStruct((NP, COUT), jnp.float32),
            jax.ShapeDtypeStruct((8, COUT), jnp.float32),
            jax.ShapeDtypeStruct((8, COUT), jnp.float32),
        ],
    )(wsum2d, wf)


def _bn_body(o_ref, sum_ref, sq_ref, g_ref, b_ref, y_ref):
    mean = sum_ref[0:1, :] * jnp.float32(1.0 / N)
    ms = sq_ref[0:1, :] * jnp.float32(1.0 / N)
    var = ms - mean * mean
    inv = 1.0 / jnp.sqrt(var + jnp.float32(BN_EPS))
    y = (o_ref[...] - mean) * inv * g_ref[0:1, :] + b_ref[0:1, :]
    y_ref[...] = jnp.where(y >= 0, y, 0.1 * y)


def _bn(out0, sums, sqs, g8, b8):
    return pl.pallas_call(
        _bn_body,
        grid=(NP // BQ,),
        in_specs=[
            pl.BlockSpec((BQ, COUT), lambda i: (i, 0)),
            pl.BlockSpec((8, COUT), lambda i: (0, 0)),
            pl.BlockSpec((8, COUT), lambda i: (0, 0)),
            pl.BlockSpec((8, COUT), lambda i: (0, 0)),
            pl.BlockSpec((8, COUT), lambda i: (0, 0)),
        ],
        out_specs=pl.BlockSpec((BQ, COUT), lambda i: (i, 0)),
        out_shape=jax.ShapeDtypeStruct((NP, COUT), jnp.float32),
    )(out0, sums, sqs, g8, b8)


def kernel(pos, x, kernel_points, W, gamma, beta):
    xyz = pos[:, 1:]
    ppad = jnp.pad(xyz, ((0, NP - N), (0, 0)), constant_values=1e9)
    qb = ppad.astype(jnp.bfloat16)
    sq = jnp.sum(ppad ** 2, axis=1)

    d2 = _d2mat(qb, qb.T, sq)

    pshadow = jnp.pad(xyz, ((0, NP - N), (0, 0)), constant_values=1e6)
    xq = jnp.pad(x.astype(jnp.bfloat16).astype(jnp.float32),
                 ((0, NP - N), (0, 0)))
    xfull = jnp.concatenate(
        [xq, pshadow, jnp.zeros((NP, ROWW - CIN - 3), jnp.float32)], axis=1)
    qpos = jnp.concatenate(
        [pshadow, jnp.zeros((NP, 13), jnp.float32)], axis=1)
    kp = jnp.pad(kernel_points.T, ((0, 0), (0, 1)))
    nbrs, wsum = _sc_aggregate(d2, xfull, qpos, kp)

    wf = jnp.pad(W, ((0, 1), (0, 0), (0, 0))).reshape(
        (KP + 1) * CIN, COUT).astype(jnp.bfloat16)
    out0, sums, sqs = _mm_stats(wsum.reshape(NP, (KP + 1) * CIN), wf)

    g8 = jnp.broadcast_to(gamma[None, :], (8, COUT))
    b8 = jnp.broadcast_to(beta[None, :], (8, COUT))
    y = _bn(out0, sums, sqs, g8, b8)

    return (pos, y[:N], nbrs[:N])


# speedup vs baseline: 1.0021x; 1.0021x over previous
import jax
import jax.numpy as jnp
from jax import lax
from jax.experimental import pallas as pl
from jax.experimental.pallas import tpu as pltpu
from jax.experimental.pallas import tpu_sc as plsc

N = 10000
NP = 10240
KN = 32
KP = 15
CIN = 128
COUT = 128
BQ = 256
SIGMA = 1.0
PREV_GRID = 0.1
RADIUS = 2.5 * SIGMA * PREV_GRID
POINT_INFLUENCE = PREV_GRID * SIGMA
BN_EPS = 1e-5

NW = 32
QPW = NP // NW


def _d2_body(qb_ref, sbt_ref, q2_ref, s2_ref, d2_ref):
    dot = lax.dot_general(qb_ref[...], sbt_ref[...], (((1,), (0,)), ((), ())),
                          preferred_element_type=jnp.float32)
    d2_ref[...] = q2_ref[...] + s2_ref[...] - 2.0 * dot


def _d2mat(qb, sbt, sq):
    return pl.pallas_call(
        _d2_body,
        grid=(NP // BQ,),
        in_specs=[
            pl.BlockSpec((BQ, 3), lambda i: (i, 0)),
            pl.BlockSpec((3, NP), lambda i: (0, 0)),
            pl.BlockSpec((BQ, 1), lambda i: (i, 0)),
            pl.BlockSpec((1, NP), lambda i: (0, 0)),
        ],
        out_specs=pl.BlockSpec((BQ, NP), lambda i: (i, 0)),
        out_shape=jax.ShapeDtypeStruct((NP, NP), jnp.float32),
    )(qb, sbt, sq[:, None], sq[None, :])


def _bf16_round(f):
    u = lax.bitcast_convert_type(f, jnp.int32)
    bias = 0x7FFF + (lax.shift_right_logical(u, 16) & 1)
    u = (u + bias) & jnp.int32(-65536)
    return lax.bitcast_convert_type(u, jnp.float32)


def _nr_sqrt(sq):
    i = lax.bitcast_convert_type(sq, jnp.int32)
    i = 0x5F3759DF - lax.shift_right_logical(i, 1)
    y = lax.bitcast_convert_type(i, jnp.float32)
    for _ in range(3):
        y = y * (1.5 - 0.5 * sq * y * y)
    return sq * y


ROWW = 144
NG = NP // 16
SEGA = 528
CAP2 = 176
LO = -0.07
NB_BINS = 64
DELTA = (RADIUS * RADIUS - LO) / NB_BINS
R2 = RADIUS * RADIUS


def _sc_body(d2_hbm, xfull_hbm, qpos_hbm, kp_hbm, nbr_hbm, wsum_hbm,
             qpos_v, kp_v, d2r_v, cand_v, candi_v, c2v_v, c2i_v, hist_v,
             idx_v, rows_v, wbuf_v, acc_v, nbub_v, semd, semg):
    c = lax.axis_index("c")
    s = lax.axis_index("s")
    wid = s * 2 + c
    base = wid * QPW
    pltpu.sync_copy(qpos_hbm.at[pl.ds(base, QPW)], qpos_v)
    pltpu.sync_copy(kp_hbm, kp_v)
    zero16 = jnp.zeros((16,), jnp.float32)
    for cc in range(8):
        acc_v[KP, pl.ds(cc * 16, 16)] = zero16
    inv_pi = jnp.float32(POINT_INFLUENCE)
    lane16 = jnp.arange(16, dtype=jnp.int32)
    r2f = jnp.float32(R2)
    inf16 = jnp.full((16,), jnp.inf, jnp.float32)

    pltpu.async_copy(d2_hbm.at[base], d2r_v.at[pl.ds(0, NP)], semd)

    def qbody(m, carry):
        buf = lax.rem(m, 2)

        @pl.when(m < QPW)
        def _select():
            doff = buf * NP
            pltpu.make_async_copy(d2_hbm.at[base],
                                  d2r_v.at[pl.ds(0, NP)], semd).wait()
            mn = jnp.minimum(m + 1, QPW - 1)
            pltpu.async_copy(d2_hbm.at[base + mn],
                             d2r_v.at[pl.ds((1 - buf) * NP, NP)], semd)

            def abody(g, offs):
                o = list(offs)
                for q in range(2):
                    gg = q * (NG // 2) + g
                    v = d2r_v[pl.ds(doff + gg * 16, 16)]
                    mask = v <= r2f
                    plsc.store_compressed(
                        cand_v.at[pl.ds(q * SEGA + o[q], 16)], v, mask=mask)
                    iv = lane16 + gg * 16
                    plsc.store_compressed(
                        candi_v.at[pl.ds(q * SEGA + o[q], 16)], iv, mask=mask)
                    cnt = plsc.all_reduce_population_count(mask)
                    cnt = cnt[0] if cnt.ndim else cnt
                    o[q] = jnp.minimum(o[q] + cnt, SEGA - 16)
                return tuple(o)

            offs = lax.fori_loop(0, NG // 2, abody,
                                 (jnp.int32(0), jnp.int32(0)), unroll=2)
            cand_v[pl.ds(offs[0], 16)] = inf16
            cand_v[pl.ds(SEGA + offs[1], 16)] = inf16

            for hh in range(NB_BINS // 16):
                hist_v[pl.ds(hh * 16, 16)] = jnp.zeros((16,), jnp.int32)

            for q in range(2):
                offq = offs[q]

                def bbody(g, carry2, _q=q, _offq=offq):
                    v = cand_v[pl.ds(_q * SEGA + g * 16, 16)]
                    valid = (lane16 + g * 16) < _offq
                    bi = ((v - jnp.float32(LO)) * jnp.float32(1.0 / DELTA))
                    bi = jnp.clip(bi.astype(jnp.int32), 0, NB_BINS - 1)
                    bi = jnp.where(valid, bi, NB_BINS - 1)
                    plsc.addupdate_scatter(hist_v, [bi],
                                           jnp.ones((16,), jnp.int32))
                    return carry2

                lax.fori_loop(0, (offq + 15) // 16, bbody, 0)

            cum = jnp.int32(0)
            bstar = jnp.int32(NB_BINS)
            for hh in range(NB_BINS // 16):
                ch = plsc.cumsum(hist_v[pl.ds(hh * 16, 16)]) + cum
                bh = jnp.where(ch >= KN, lane16 + hh * 16, NB_BINS)
                bstar = jnp.minimum(bstar, jnp.min(bh))
                cum = ch[15]

            off2 = jnp.int32(0)
            for q in range(2):
                offq = offs[q]

                def cbody(g, o2, _q=q, _offq=offq):
                    v = cand_v[pl.ds(_q * SEGA + g * 16, 16)]
                    iv2 = candi_v[pl.ds(_q * SEGA + g * 16, 16)]
                    valid = (lane16 + g * 16) < _offq
                    bi = ((v - jnp.float32(LO)) * jnp.float32(1.0 / DELTA))
                    bi = jnp.clip(bi.astype(jnp.int32), 0, NB_BINS - 1)
                    m2 = (bi <= bstar) & valid
                    plsc.store_compressed(c2v_v.at[pl.ds(o2, 16)], v,
                                          mask=m2)
                    plsc.store_compressed(c2i_v.at[pl.ds(o2, 16)], iv2,
                                          mask=m2)
                    cnt = plsc.all_reduce_population_count(m2)
                    cnt = cnt[0] if cnt.ndim else cnt
                    return jnp.minimum(o2 + cnt, CAP2 - 32)

                off2 = lax.fori_loop(0, (offq + 15) // 16, cbody, off2)

            c2v_v[pl.ds(off2, 16)] = inf16
            nv2 = (off2 + 15) // 16

            lastm = jnp.full((16,), -jnp.inf, jnp.float32)
            lasti = jnp.full((16,), -1, jnp.int32)
            nb0 = jnp.full((16,), N, jnp.int32)
            nb1 = jnp.full((16,), N, jnp.int32)
            for t in range(KN):
                def dbody(g, mm):
                    mval, midx = mm
                    v = c2v_v[pl.ds(g * 16, 16)]
                    ix = c2i_v[pl.ds(g * 16, 16)]
                    elig = (v > lastm) | ((v == lastm) & (ix > lasti))
                    bet = elig & ((v < mval) | ((v == mval) & (ix < midx)))
                    return (jnp.where(bet, v, mval),
                            jnp.where(bet, ix, midx))

                mval, midx = lax.fori_loop(
                    0, nv2, dbody,
                    (inf16, jnp.full((16,), NP, jnp.int32)))
                m_sc = jnp.min(mval)
                i_sc = jnp.min(jnp.where(mval == m_sc, midx, NP))
                nb_sc = jnp.where(m_sc <= r2f, i_sc, N)
                if t < 16:
                    nb0 = jnp.where(lane16 == t, nb_sc, nb0)
                else:
                    nb1 = jnp.where(lane16 == (t - 16), nb_sc, nb1)
                lastm = jnp.full((16,), m_sc, jnp.float32)
                lasti = jnp.full((16,), i_sc, jnp.int32)

            nbub_v[pl.ds(0, 16)] = nb0
            nbub_v[pl.ds(16, 16)] = nb1
            pltpu.sync_copy(nbub_v, nbr_hbm.at[base + m])
            idx_v[pl.ds(buf * KN, 16)] = nb0
            idx_v[pl.ds(buf * KN + 16, 16)] = nb1

        @pl.when(m > 0)
        def _agg():
            aoff = (1 - buf) * KN
            pltpu.make_async_copy(
                xfull_hbm.at[idx_v.at[pl.ds(aoff, KN)]],
                rows_v.at[pl.ds(aoff, KN)], semg).wait()
            qrow = qpos_v[m - 1, pl.ds(0, 16)]
            qx = qrow[0]
            qy = qrow[1]
            qz = qrow[2]
            kpx = kp_v[0, :]
            kpy = kp_v[1, :]
            kpz = kp_v[2, :]

            def wbody(n, carry2):
                pv = rows_v[aoff + n, pl.ds(CIN, 16)]
                ax = (pv[0] - qx) - kpx
                ay = (pv[1] - qy) - kpy
                az = (pv[2] - qz) - kpz
                sq2 = ax * ax + ay * ay + az * az
                sr = _nr_sqrt(sq2)
                w = jnp.maximum(1.0 - sr / inv_pi, 0.0)
                wbuf_v[n, pl.ds(0, 16)] = _bf16_round(w)
                return carry2

            lax.fori_loop(0, KN, wbody, 0, unroll=4)

            for kg in range(3):
                def nbody(n, accs):
                    rr = [rows_v[aoff + n, pl.ds(cc * 16, 16)]
                          for cc in range(8)]
                    wv = wbuf_v[n, pl.ds(0, 16)]
                    out = list(accs)
                    for j in range(5):
                        wk = wv[kg * 5 + j]
                        for cc in range(8):
                            out[j * 8 + cc] = out[j * 8 + cc] + wk * rr[cc]
                    return tuple(out)

                accs = lax.fori_loop(0, KN, nbody,
                                     tuple(jnp.zeros((16,), jnp.float32)
                                           for _ in range(40)),
                                     unroll=2)
                for j in range(5):
                    for cc in range(8):
                        acc_v[kg * 5 + j, pl.ds(cc * 16, 16)] = \
                            accs[j * 8 + cc]
            pltpu.sync_copy(acc_v, wsum_hbm.at[base + m - 1])

        @pl.when(m < QPW)
        def _issue_gather():
            pltpu.async_copy(
                xfull_hbm.at[idx_v.at[pl.ds(buf * KN, KN)]],
                rows_v.at[pl.ds(buf * KN, KN)], semg)

        return carry

    lax.fori_loop(0, QPW + 1, qbody, 0)
    pltpu.make_async_copy(d2_hbm.at[base],
                          d2r_v.at[pl.ds(0, NP)], semd).wait()


def _sc_aggregate(d2, xfull, qpos, kp):
    mesh = plsc.VectorSubcoreMesh(core_axis_name="c", subcore_axis_name="s")
    f = pl.kernel(
        _sc_body,
        out_type=[
            jax.ShapeDtypeStruct((NP, KN), jnp.int32),
            jax.ShapeDtypeStruct((NP, KP + 1, CIN), jnp.float32),
        ],
        mesh=mesh,
        compiler_params=pltpu.CompilerParams(use_tc_tiling_on_sc=False,
                                             needs_layout_passes=False),
        scratch_types=[
            pltpu.VMEM((QPW, 16), jnp.float32),
            pltpu.VMEM((3, 16), jnp.float32),
            pltpu.VMEM((2 * NP,), jnp.float32),
            pltpu.VMEM((2 * SEGA,), jnp.float32),
            pltpu.VMEM((2 * SEGA,), jnp.int32),
            pltpu.VMEM((CAP2,), jnp.float32),
            pltpu.VMEM((CAP2,), jnp.int32),
            pltpu.VMEM((NB_BINS,), jnp.int32),
            pltpu.VMEM((2 * KN,), jnp.int32),
            pltpu.VMEM((2 * KN, ROWW), jnp.float32),
            pltpu.VMEM((KN, 16), jnp.float32),
            pltpu.VMEM((KP + 1, CIN), jnp.float32),
            pltpu.VMEM((KN,), jnp.int32),
            pltpu.SemaphoreType.DMA,
            pltpu.SemaphoreType.DMA,
        ],
    )
    return f(d2, xfull, qpos, kp)


def _mm_body(ws_ref, wf_ref, out_ref, sum_ref, sq_ref):
    i = pl.program_id(0)
    a = ws_ref[...].astype(jnp.bfloat16)
    o = lax.dot_general(a, wf_ref[...], (((1,), (0,)), ((), ())),
                        preferred_element_type=jnp.float32)
    out_ref[...] = o
    rowid = i * BQ + lax.broadcasted_iota(jnp.int32, (BQ, 1), 0)
    om = jnp.where(rowid < N, o, 0.0)
    ps = jnp.sum(om, axis=0, keepdims=True)
    pq = jnp.sum(om * om, axis=0, keepdims=True)

    @pl.when(i == 0)
    def _():
        sum_ref[...] = jnp.zeros_like(sum_ref)
        sq_ref[...] = jnp.zeros_like(sq_ref)

    sum_ref[...] += jnp.broadcast_to(ps, (8, COUT))
    sq_ref[...] += jnp.broadcast_to(pq, (8, COUT))


def _mm_stats(wsum2d, wf):
    return pl.pallas_call(
        _mm_body,
        grid=(NP // BQ,),
        in_specs=[
            pl.BlockSpec((BQ, (KP + 1) * CIN), lambda i: (i, 0)),
            pl.BlockSpec(((KP + 1) * CIN, COUT), lambda i: (0, 0)),
        ],
        out_specs=[
            pl.BlockSpec((BQ, COUT), lambda i: (i, 0)),
            pl.BlockSpec((8, COUT), lambda i: (0, 0)),
            pl.BlockSpec((8, COUT), lambda i: (0, 0)),
        ],
        out_shape=[
            jax.ShapeDtypeStruct((NP, COUT), jnp.float32),
            jax.ShapeDtypeStruct((8, COUT), jnp.float32),
            jax.ShapeDtypeStruct((8, COUT), jnp.float32),
        ],
    )(wsum2d, wf)


def _bn_body(o_ref, sum_ref, sq_ref, g_ref, b_ref, y_ref):
    mean = sum_ref[0:1, :] * jnp.float32(1.0 / N)
    ms = sq_ref[0:1, :] * jnp.float32(1.0 / N)
    var = ms - mean * mean
    inv = 1.0 / jnp.sqrt(var + jnp.float32(BN_EPS))
    y = (o_ref[...] - mean) * inv * g_ref[0:1, :] + b_ref[0:1, :]
    y_ref[...] = jnp.where(y >= 0, y, 0.1 * y)


def _bn(out0, sums, sqs, g8, b8):
    return pl.pallas_call(
        _bn_body,
        grid=(NP // BQ,),
        in_specs=[
            pl.BlockSpec((BQ, COUT), lambda i: (i, 0)),
            pl.BlockSpec((8, COUT), lambda i: (0, 0)),
            pl.BlockSpec((8, COUT), lambda i: (0, 0)),
            pl.BlockSpec((8, COUT), lambda i: (0, 0)),
            pl.BlockSpec((8, COUT), lambda i: (0, 0)),
        ],
        out_specs=pl.BlockSpec((BQ, COUT), lambda i: (i, 0)),
        out_shape=jax.ShapeDtypeStruct((NP, COUT), jnp.float32),
    )(out0, sums, sqs, g8, b8)


def kernel(pos, x, kernel_points, W, gamma, beta):
    xyz = pos[:, 1:]
    ppad = jnp.pad(xyz, ((0, NP - N), (0, 0)), constant_values=1e9)
    qb = ppad.astype(jnp.bfloat16)
    sq = jnp.sum(ppad ** 2, axis=1)

    d2 = _d2mat(qb, qb.T, sq)

    pshadow = jnp.pad(xyz, ((0, NP - N), (0, 0)), constant_values=1e6)
    xq = jnp.pad(x.astype(jnp.bfloat16).astype(jnp.float32),
                 ((0, NP - N), (0, 0)))
    xfull = jnp.concatenate(
        [xq, pshadow, jnp.zeros((NP, ROWW - CIN - 3), jnp.float32)], axis=1)
    qpos = jnp.concatenate(
        [pshadow, jnp.zeros((NP, 13), jnp.float32)], axis=1)
    kp = jnp.pad(kernel_points.T, ((0, 0), (0, 1)))
    nbrs, wsum = _sc_aggregate(d2, xfull, qpos, kp)

    wf = jnp.pad(W, ((0, 1), (0, 0), (0, 0))).reshape(
        (KP + 1) * CIN, COUT).astype(jnp.bfloat16)
    out0, sums, sqs = _mm_stats(wsum.reshape(NP, (KP + 1) * CIN), wf)

    g8 = jnp.broadcast_to(gamma[None, :], (8, COUT))
    b8 = jnp.broadcast_to(beta[None, :], (8, COUT))
    y = _bn(out0, sums, sqs, g8, b8)

    return (pos, y[:N], nbrs[:N])
